# baseline (speedup 1.0000x reference)
import functools

import jax
import jax.numpy as jnp
from jax import lax
from jax.experimental import pallas as pl
from jax.experimental.pallas import tpu as pltpu
from jax.experimental.pallas import tpu_sc as plsc

NUM_EMB = 1_000_000
D = 32
NROW = 16384
HIST = 50
HPAD = 128
H2 = 56
DPAD = 128
NW = 32
ROWS_PER_W = NROW // NW
GR = 8
NCH = ROWS_PER_W // GR
NB = 4
DP = 2

_mesh = plsc.VectorSubcoreMesh(core_axis_name="c", subcore_axis_name="s")


@functools.partial(
    pl.kernel,
    mesh=_mesh,
    out_type=jax.ShapeDtypeStruct((NROW, H2, DPAD), jnp.float32),
    scratch_types=[
        pltpu.VMEM((ROWS_PER_W, HPAD), jnp.int32),
        [pltpu.VMEM((GR, H2, D), jnp.float32) for _ in range(NB)],
        [pltpu.SemaphoreType.DMA for _ in range(NB)],
        [pltpu.SemaphoreType.DMA for _ in range(NB)],
    ],
    compiler_params=pltpu.CompilerParams(use_tc_tiling_on_sc=False),
)
def _gather(idx_hbm, table_hbm, out_hbm, idx_all, rows, gsem, ssem):
    wid = lax.axis_index("s") * 2 + lax.axis_index("c")
    base = wid * ROWS_PER_W

    pltpu.sync_copy(idx_hbm.at[pl.ds(base, ROWS_PER_W)], idx_all)

    def row_gather_descr(c, r, slot):
        src = table_hbm.at[idx_all.at[c * GR + r, pl.ds(0, H2)]]
        return pltpu.make_async_copy(src, rows[slot].at[r], gsem[slot])

    def store_descr(c, slot):
        dst = out_hbm.at[pl.ds(base + c * GR, GR), :, pl.ds(0, D)]
        return pltpu.make_async_copy(rows[slot], dst, ssem[slot])

    for b in range(DP):
        for r in range(GR):
            row_gather_descr(b, r, b).start()

    @pl.loop(0, NCH, step=NB)
    def _(j):
        for b in range(NB):
            c = j + b
            cp = c + DP
            pb = (b + DP) % NB

            @pl.when(cp < NCH)
            def _():
                @pl.when(cp >= NB)
                def _():
                    store_descr(cp - NB, pb).wait()

                for r in range(GR):
                    row_gather_descr(cp, r, pb).start()

            for r in range(GR):
                row_gather_descr(c, r, b).wait()
            store_descr(c, b).start()

    for b in range(NB):
        store_descr(NCH - NB + b, b).wait()


def kernel(input, table):
    idx = jnp.pad(input.astype(jnp.int32), ((0, 0), (0, HPAD - HIST)))
    out = _gather(idx, table)
    return out[:, :HIST, :D]


# speedup vs baseline: 1.5884x; 1.5884x over previous
import functools

import jax
import jax.numpy as jnp
from jax import lax
from jax.experimental import pallas as pl
from jax.experimental.pallas import tpu as pltpu
from jax.experimental.pallas import tpu_sc as plsc

NUM_EMB = 1_000_000
D = 32
NROW = 16384
HIST = 50
NW = 32
BW = 128
DT = D // 8
BT_PER_W = NROW // BW // NW
NU = HIST * BT_PER_W
NB = 4
DP = 2

_mesh = plsc.VectorSubcoreMesh(core_axis_name="c", subcore_axis_name="s")


@functools.partial(
    pl.kernel,
    mesh=_mesh,
    out_type=jax.ShapeDtypeStruct((HIST, DT, NROW // BW, 8, BW), jnp.float32),
    scratch_types=[
        pltpu.VMEM((HIST, BW * BT_PER_W), jnp.int32),
        [pltpu.VMEM((BW, D), jnp.float32) for _ in range(NB)],
        [pltpu.VMEM((DT, 8, BW), jnp.float32) for _ in range(NB)],
        [pltpu.SemaphoreType.DMA for _ in range(NB)],
        [pltpu.SemaphoreType.DMA for _ in range(NB)],
    ],
    compiler_params=pltpu.CompilerParams(
        use_tc_tiling_on_sc=False, needs_layout_passes=False
    ),
)
def _gather(idx_hbm, table_hbm, out_hbm, idx_v, gbuf, tbuf, gsem, ssem):
    wid = lax.axis_index("s") * 2 + lax.axis_index("c")
    base_b = wid * (BW * BT_PER_W)

    pltpu.sync_copy(idx_hbm.at[:, pl.ds(base_b, BW * BT_PER_W)], idx_v)

    iota16 = lax.iota(jnp.int32, 16)

    def gather_descr(u, slot):
        h = u // BT_PER_W
        vb = u % BT_PER_W
        src = table_hbm.at[idx_v.at[h, pl.ds(vb * BW, BW)]]
        return pltpu.make_async_copy(src, gbuf[slot], gsem[slot])

    def store_descrs(u, slot):
        h = u // BT_PER_W
        bt = wid * BT_PER_W + (u % BT_PER_W)
        return [
            pltpu.make_async_copy(
                tbuf[slot].at[dt], out_hbm.at[h, dt, bt], ssem[slot]
            )
            for dt in range(DT)
        ]

    def transpose(slot):
        for dt in range(DT):
            for f in range(8):
                col = jnp.full((16,), dt * 8 + f, jnp.int32)
                for g in range(BW // 16):
                    v = plsc.load_gather(gbuf[slot], [iota16 + g * 16, col])
                    tbuf[slot][dt, f, pl.ds(g * 16, 16)] = v

    for b in range(DP):
        gather_descr(b, b).start()

    @pl.loop(0, NU, step=NB)
    def _(j):
        for b in range(NB):
            u = j + b
            up = u + DP
            pb = (b + DP) % NB

            @pl.when(up < NU)
            def _():
                gather_descr(up, pb).start()

            gather_descr(u, b).wait()

            @pl.when(u >= NB)
            def _():
                for d in store_descrs(u - NB, b):
                    d.wait()

            transpose(b)
            for d in store_descrs(u, b):
                d.start()

    for b in range(NB):
        for d in store_descrs(NU - NB + b, b):
            d.wait()


def kernel(input, table):
    out5 = _gather(input.astype(jnp.int32).T, table)
    return out5.transpose(2, 4, 0, 1, 3).reshape(NROW, HIST, D)


# speedup vs baseline: 1.5886x; 1.0001x over previous
import functools

import jax
import jax.numpy as jnp
from jax import lax
from jax.experimental import pallas as pl
from jax.experimental.pallas import tpu as pltpu
from jax.experimental.pallas import tpu_sc as plsc

NUM_EMB = 1_000_000
D = 32
NROW = 16384
HIST = 50
NW = 32
BW = 128
DT = D // 8
BT_PER_W = NROW // BW // NW
NU = HIST * BT_PER_W
NB = 4
DP = 2

_mesh = plsc.VectorSubcoreMesh(core_axis_name="c", subcore_axis_name="s")


@functools.partial(
    pl.kernel,
    mesh=_mesh,
    out_type=jax.ShapeDtypeStruct((HIST, DT, NROW // BW, 8, BW), jnp.float32),
    scratch_types=[
        pltpu.VMEM((HIST, BW * BT_PER_W), jnp.int32),
        [pltpu.VMEM((BW, D), jnp.float32) for _ in range(NB)],
        [pltpu.VMEM((DT, 8, BW), jnp.float32) for _ in range(NB)],
        [pltpu.SemaphoreType.DMA for _ in range(NB)],
        [pltpu.SemaphoreType.DMA for _ in range(NB)],
    ],
    compiler_params=pltpu.CompilerParams(
        use_tc_tiling_on_sc=False, needs_layout_passes=False
    ),
)
def _gather(idx_hbm, table_hbm, out_hbm, idx_v, gbuf, tbuf, gsem, ssem):
    wid = lax.axis_index("s") * 2 + lax.axis_index("c")
    base_b = wid * (BW * BT_PER_W)

    pltpu.sync_copy(idx_hbm.at[:, pl.ds(base_b, BW * BT_PER_W)], idx_v)

    iota16 = lax.iota(jnp.int32, 16)
    row_ids = [iota16 + g * 16 for g in range(BW // 16)]
    col_ids = [jnp.full((16,), col, jnp.int32) for col in range(D)]

    def gather_descr(u, slot):
        h = u // BT_PER_W
        vb = u % BT_PER_W
        src = table_hbm.at[idx_v.at[h, pl.ds(vb * BW, BW)]]
        return pltpu.make_async_copy(src, gbuf[slot], gsem[slot])

    def store_descrs(u, slot):
        h = u // BT_PER_W
        bt = wid * BT_PER_W + (u % BT_PER_W)
        return [
            pltpu.make_async_copy(
                tbuf[slot].at[dt], out_hbm.at[h, dt, bt], ssem[slot]
            )
            for dt in range(DT)
        ]

    def transpose(slot):
        for dt in range(DT):
            for f in range(8):
                for g in range(BW // 16):
                    v = plsc.load_gather(
                        gbuf[slot], [row_ids[g], col_ids[dt * 8 + f]]
                    )
                    tbuf[slot][dt, f, pl.ds(g * 16, 16)] = v

    for b in range(DP):
        gather_descr(b, b).start()

    @pl.loop(0, NU, step=NB)
    def _(j):
        for b in range(NB):
            u = j + b
            up = u + DP
            pb = (b + DP) % NB

            @pl.when(up < NU)
            def _():
                gather_descr(up, pb).start()

            gather_descr(u, b).wait()

            @pl.when(u >= NB)
            def _():
                for d in store_descrs(u - NB, b):
                    d.wait()

            transpose(b)
            for d in store_descrs(u, b):
                d.start()

    for b in range(NB):
        for d in store_descrs(NU - NB + b, b):
            d.wait()


def kernel(input, table):
    out5 = _gather(input.astype(jnp.int32).T, table)
    return out5.transpose(2, 4, 0, 1, 3).reshape(NROW, HIST, D)


# speedup vs baseline: 1.8111x; 1.1400x over previous
import functools

import jax
import jax.numpy as jnp
from jax import lax
from jax.experimental import pallas as pl
from jax.experimental.pallas import tpu as pltpu
from jax.experimental.pallas import tpu_sc as plsc

NUM_EMB = 1_000_000
D = 32
NROW = 16384
HIST = 50
NW = 32
BW = 128
DT = D // 8
BT_PER_W = NROW // BW // NW
NU = HIST * BT_PER_W
NB = 4
DP = 2

_mesh = plsc.VectorSubcoreMesh(core_axis_name="c", subcore_axis_name="s")


@functools.partial(
    pl.kernel,
    mesh=_mesh,
    out_type=jax.ShapeDtypeStruct((HIST, DT, NROW // BW, 8, BW), jnp.float32),
    scratch_types=[
        pltpu.VMEM((HIST, BW * BT_PER_W), jnp.int32),
        [pltpu.VMEM((BW, D), jnp.float32) for _ in range(NB)],
        [pltpu.VMEM((DT, 8, BW), jnp.float32) for _ in range(NB)],
        [pltpu.SemaphoreType.DMA for _ in range(NB)],
        [pltpu.SemaphoreType.DMA for _ in range(NB)],
    ],
    compiler_params=pltpu.CompilerParams(
        use_tc_tiling_on_sc=False, needs_layout_passes=False
    ),
)
def _gather(idx_hbm, table_hbm, out_hbm, idx_v, gbuf, tbuf, gsem, ssem):
    wid = lax.axis_index("s") * 2 + lax.axis_index("c")
    base_b = wid * (BW * BT_PER_W)

    pltpu.sync_copy(idx_hbm.at[:, pl.ds(base_b, BW * BT_PER_W)], idx_v)

    iota16 = lax.iota(jnp.int32, 16)
    row_ids = [iota16 + g * 16 for g in range(BW // 16)]
    col_ids = [jnp.full((16,), col, jnp.int32) for col in range(D)]

    def gather_descr(u, slot):
        h = u // BT_PER_W
        vb = u % BT_PER_W
        src = table_hbm.at[idx_v.at[h, pl.ds(vb * BW, BW)]]
        return pltpu.make_async_copy(src, gbuf[slot], gsem[slot])

    def store_descrs(u, slot):
        h = u // BT_PER_W
        bt = wid * BT_PER_W + (u % BT_PER_W)
        return [
            pltpu.make_async_copy(
                tbuf[slot].at[dt], out_hbm.at[h, dt, bt], ssem[slot]
            )
            for dt in range(DT)
        ]

    def transpose(slot):
        for dt in range(DT):
            for f in range(8):
                vs = [
                    plsc.load_gather(
                        gbuf[slot], [row_ids[g], col_ids[dt * 8 + f]]
                    )
                    for g in range(BW // 16)
                ]
                for g in range(BW // 16):
                    tbuf[slot][dt, f, pl.ds(g * 16, 16)] = vs[g]

    for b in range(DP):
        gather_descr(b, b).start()

    @pl.loop(0, NU, step=NB)
    def _(j):
        for b in range(NB):
            u = j + b
            up = u + DP
            pb = (b + DP) % NB

            @pl.when(up < NU)
            def _():
                gather_descr(up, pb).start()

            gather_descr(u, b).wait()

            @pl.when(u >= NB)
            def _():
                for d in store_descrs(u - NB, b):
                    d.wait()

            transpose(b)
            for d in store_descrs(u, b):
                d.start()

    for b in range(NB):
        for d in store_descrs(NU - NB + b, b):
            d.wait()


def kernel(input, table):
    out5 = _gather(input.astype(jnp.int32).T, table)
    return out5.transpose(2, 4, 0, 1, 3).reshape(NROW, HIST, D)


# speedup vs baseline: 1.8191x; 1.0044x over previous
import functools

import jax
import jax.numpy as jnp
from jax import lax
from jax.experimental import pallas as pl
from jax.experimental.pallas import tpu as pltpu
from jax.experimental.pallas import tpu_sc as plsc

NUM_EMB = 1_000_000
D = 32
NROW = 16384
HIST = 50
NW = 32
BW = 128
DT = D // 8
BT_PER_W = NROW // BW // NW
NU = HIST * BT_PER_W
NB = 4
DP = 2

_mesh = plsc.VectorSubcoreMesh(core_axis_name="c", subcore_axis_name="s")


@functools.partial(
    pl.kernel,
    mesh=_mesh,
    out_type=jax.ShapeDtypeStruct((HIST, DT, NROW // BW, 8, BW), jnp.float32),
    scratch_types=[
        pltpu.VMEM((HIST, BW * BT_PER_W), jnp.int32),
        [pltpu.VMEM((BW, D), jnp.float32) for _ in range(NB)],
        [pltpu.VMEM((DT, 8, BW), jnp.float32) for _ in range(NB)],
        [pltpu.SemaphoreType.DMA for _ in range(NB)],
        [pltpu.SemaphoreType.DMA for _ in range(NB)],
    ],
    compiler_params=pltpu.CompilerParams(
        use_tc_tiling_on_sc=False, needs_layout_passes=False
    ),
)
def _gather(idx_hbm, table_hbm, out_hbm, idx_v, gbuf, tbuf, gsem, ssem):
    wid = lax.axis_index("s") * 2 + lax.axis_index("c")
    base_b = wid * (BW * BT_PER_W)

    pltpu.sync_copy(idx_hbm.at[:, pl.ds(base_b, BW * BT_PER_W)], idx_v)

    iota16 = lax.iota(jnp.int32, 16)
    row_ids = [iota16 + g * 16 for g in range(BW // 16)]
    col_ids = [jnp.full((16,), col, jnp.int32) for col in range(D)]

    def gather_descr(u, slot):
        h = u // BT_PER_W
        vb = u % BT_PER_W
        src = table_hbm.at[idx_v.at[h, pl.ds(vb * BW, BW)]]
        return pltpu.make_async_copy(src, gbuf[slot], gsem[slot])

    def store_descrs(u, slot):
        h = u // BT_PER_W
        bt = wid * BT_PER_W + (u % BT_PER_W)
        return [
            pltpu.make_async_copy(
                tbuf[slot].at[dt], out_hbm.at[h, dt, bt], ssem[slot]
            )
            for dt in range(DT)
        ]

    def transpose(slot):
        groups = [(dt, f) for dt in range(DT) for f in range(8)]

        def loads(k):
            dt, f = groups[k]
            return [
                plsc.load_gather(gbuf[slot], [row_ids[g], col_ids[dt * 8 + f]])
                for g in range(BW // 16)
            ]

        def stores(k, vs):
            dt, f = groups[k]
            for g in range(BW // 16):
                tbuf[slot][dt, f, pl.ds(g * 16, 16)] = vs[g]

        prev = loads(0)
        for k in range(1, len(groups)):
            cur = loads(k)
            stores(k - 1, prev)
            prev = cur
        stores(len(groups) - 1, prev)

    for b in range(DP):
        gather_descr(b, b).start()

    @pl.loop(0, NU, step=NB)
    def _(j):
        for b in range(NB):
            u = j + b
            up = u + DP
            pb = (b + DP) % NB

            @pl.when(up < NU)
            def _():
                gather_descr(up, pb).start()

            gather_descr(u, b).wait()

            @pl.when(u >= NB)
            def _():
                for d in store_descrs(u - NB, b):
                    d.wait()

            transpose(b)
            for d in store_descrs(u, b):
                d.start()

    for b in range(NB):
        for d in store_descrs(NU - NB + b, b):
            d.wait()


def kernel(input, table):
    out5 = _gather(input.astype(jnp.int32).T, table)
    return out5.transpose(2, 4, 0, 1, 3).reshape(NROW, HIST, D)


# speedup vs baseline: 2.4048x; 1.3220x over previous
import functools

import jax
import jax.numpy as jnp
from jax import lax
from jax.experimental import pallas as pl
from jax.experimental.pallas import tpu as pltpu
from jax.experimental.pallas import tpu_sc as plsc

NUM_EMB = 1_000_000
D = 32
NROW = 16384
HIST = 50
NW = 32
BW = 128
DT = D // 8
BT_PER_W = NROW // BW // NW
NU = HIST * BT_PER_W
NB = 4
DP = 2

_mesh = plsc.VectorSubcoreMesh(core_axis_name="c", subcore_axis_name="s")


@functools.partial(
    pl.kernel,
    mesh=_mesh,
    out_type=jax.ShapeDtypeStruct((HIST, DT, NROW // BW, 8, BW), jnp.float32),
    scratch_types=[
        pltpu.VMEM((HIST, BW * BT_PER_W), jnp.int32),
        [pltpu.VMEM((BW, D), jnp.float32) for _ in range(NB)],
        [pltpu.VMEM((DT, 8, BW + 1), jnp.float32) for _ in range(NB)],
        [pltpu.SemaphoreType.DMA for _ in range(NB)],
        [pltpu.SemaphoreType.DMA for _ in range(NB)],
    ],
    compiler_params=pltpu.CompilerParams(
        use_tc_tiling_on_sc=False, needs_layout_passes=False
    ),
)
def _gather(idx_hbm, table_hbm, out_hbm, idx_v, gbuf, tbuf, gsem, ssem):
    wid = lax.axis_index("s") * 2 + lax.axis_index("c")
    base_b = wid * (BW * BT_PER_W)

    pltpu.sync_copy(idx_hbm.at[:, pl.ds(base_b, BW * BT_PER_W)], idx_v)

    iota16 = lax.iota(jnp.int32, 16)
    dt_ids = [(cc * 16 + iota16) // 8 for cc in range(D // 16)]
    f_ids = [(cc * 16 + iota16) % 8 for cc in range(D // 16)]

    def gather_descr(u, slot):
        h = u // BT_PER_W
        vb = u % BT_PER_W
        src = table_hbm.at[idx_v.at[h, pl.ds(vb * BW, BW)]]
        return pltpu.make_async_copy(src, gbuf[slot], gsem[slot])

    def store_descrs(u, slot):
        h = u // BT_PER_W
        bt = wid * BT_PER_W + (u % BT_PER_W)
        return [
            pltpu.make_async_copy(
                tbuf[slot].at[dt, :, pl.ds(0, BW)],
                out_hbm.at[h, dt, bt],
                ssem[slot],
            )
            for dt in range(DT)
        ]

    def transpose(slot):
        for c in range(BW):
            cv = jnp.full((16,), c, jnp.int32)
            for cc in range(D // 16):
                x = gbuf[slot][c, pl.ds(cc * 16, 16)]
                plsc.store_scatter(
                    tbuf[slot], [dt_ids[cc], f_ids[cc], cv], x
                )

    for b in range(DP):
        gather_descr(b, b).start()

    @pl.loop(0, NU, step=NB)
    def _(j):
        for b in range(NB):
            u = j + b
            up = u + DP
            pb = (b + DP) % NB

            @pl.when(up < NU)
            def _():
                gather_descr(up, pb).start()

            gather_descr(u, b).wait()

            @pl.when(u >= NB)
            def _():
                for d in store_descrs(u - NB, b):
                    d.wait()

            transpose(b)
            for d in store_descrs(u, b):
                d.start()

    for b in range(NB):
        for d in store_descrs(NU - NB + b, b):
            d.wait()


def kernel(input, table):
    out5 = _gather(input.astype(jnp.int32).T, table)
    return out5.transpose(2, 4, 0, 1, 3).reshape(NROW, HIST, D)


# speedup vs baseline: 2.4060x; 1.0005x over previous
import functools

import jax
import jax.numpy as jnp
from jax import lax
from jax.experimental import pallas as pl
from jax.experimental.pallas import tpu as pltpu
from jax.experimental.pallas import tpu_sc as plsc

NUM_EMB = 1_000_000
D = 32
NROW = 16384
HIST = 50
NW = 32
BW = 128
DT = D // 8
BT_PER_W = NROW // BW // NW
NU = HIST * BT_PER_W
NB = 4
DP = 2

_mesh = plsc.VectorSubcoreMesh(core_axis_name="c", subcore_axis_name="s")


@functools.partial(
    pl.kernel,
    mesh=_mesh,
    out_type=jax.ShapeDtypeStruct((HIST, DT, NROW // BW, 8, BW), jnp.float32),
    scratch_types=[
        pltpu.VMEM((HIST, BW * BT_PER_W), jnp.int32),
        [pltpu.VMEM((BW, D), jnp.float32) for _ in range(NB)],
        [pltpu.VMEM((D, BW + 1), jnp.float32) for _ in range(NB)],
        [pltpu.SemaphoreType.DMA for _ in range(NB)],
        [pltpu.SemaphoreType.DMA for _ in range(NB)],
    ],
    compiler_params=pltpu.CompilerParams(
        use_tc_tiling_on_sc=False, needs_layout_passes=False
    ),
)
def _gather(idx_hbm, table_hbm, out_hbm, idx_v, gbuf, tbuf, gsem, ssem):
    wid = lax.axis_index("s") * 2 + lax.axis_index("c")
    base_b = wid * (BW * BT_PER_W)

    pltpu.sync_copy(idx_hbm.at[:, pl.ds(base_b, BW * BT_PER_W)], idx_v)

    iota16 = lax.iota(jnp.int32, 16)
    col_ids = [cc * 16 + iota16 for cc in range(D // 16)]

    def gather_descr(u, slot):
        h = u // BT_PER_W
        vb = u % BT_PER_W
        src = table_hbm.at[idx_v.at[h, pl.ds(vb * BW, BW)]]
        return pltpu.make_async_copy(src, gbuf[slot], gsem[slot])

    def store_descrs(u, slot):
        h = u // BT_PER_W
        bt = wid * BT_PER_W + (u % BT_PER_W)
        return [
            pltpu.make_async_copy(
                tbuf[slot].at[pl.ds(dt * 8, 8), pl.ds(0, BW)],
                out_hbm.at[h, dt, bt],
                ssem[slot],
            )
            for dt in range(DT)
        ]

    def transpose(slot):
        for c in range(BW):
            cv = jnp.full((16,), c, jnp.int32)
            for cc in range(D // 16):
                x = gbuf[slot][c, pl.ds(cc * 16, 16)]
                plsc.store_scatter(tbuf[slot], [col_ids[cc], cv], x)

    for b in range(DP):
        gather_descr(b, b).start()

    @pl.loop(0, NU, step=NB)
    def _(j):
        for b in range(NB):
            u = j + b
            up = u + DP
            pb = (b + DP) % NB

            @pl.when(up < NU)
            def _():
                gather_descr(up, pb).start()

            gather_descr(u, b).wait()

            @pl.when(u >= NB)
            def _():
                for d in store_descrs(u - NB, b):
                    d.wait()

            transpose(b)
            for d in store_descrs(u, b):
                d.start()

    for b in range(NB):
        for d in store_descrs(NU - NB + b, b):
            d.wait()


def kernel(input, table):
    out5 = _gather(input.astype(jnp.int32).T, table)
    return out5.transpose(2, 4, 0, 1, 3).reshape(NROW, HIST, D)
